# baseline (speedup 1.0000x reference)
import jax
import jax.numpy as jnp
from jax.experimental import pallas as pl

_K = 32


def _encode_body(x_ref, w_ref, b_ref, a_ref):
    z = jax.lax.dot_general(
        x_ref[...], w_ref[...], (((1,), (1,)), ((), ())),
        preferred_element_type=jnp.float32)
    z = z + b_ref[...]
    a_ref[...] = jnp.where(z > 0.0, z, 0.0)


def _select_decode_body(a_ref, wdt_ref, bd_ref, sp_ref, xh_ref):
    a = a_ref[...]
    ai = jax.lax.bitcast_convert_type(a, jnp.int32)
    rows = a.shape[0]

    def bit_step(b, t):
        cand = t | jax.lax.shift_left(1, 30 - b)
        cnt = jnp.sum((ai >= cand).astype(jnp.int32), axis=1, keepdims=True)
        return jnp.where(cnt >= _K, cand, t)

    t = jax.lax.fori_loop(0, 31, bit_step, jnp.zeros((rows, 1), jnp.int32))
    s = jnp.where(ai >= t, a, 0.0)
    sp_ref[...] = s
    xh = jax.lax.dot_general(
        s.astype(jnp.bfloat16), wdt_ref[...], (((1,), (0,)), ((), ())),
        preferred_element_type=jnp.float32)
    xh_ref[...] = xh + bd_ref[...]


def kernel(x, W_enc, b_enc, W_dec, b_dec):
    n, d_model = x.shape
    d_dict = W_enc.shape[0]
    bre = min(512, n)
    bc = min(2048, d_dict)
    br2 = min(256, n)

    a = pl.pallas_call(
        _encode_body,
        grid=(d_dict // bc, n // bre),
        in_specs=[
            pl.BlockSpec((bre, d_model), lambda cb, rb: (rb, 0)),
            pl.BlockSpec((bc, d_model), lambda cb, rb: (cb, 0)),
            pl.BlockSpec((1, bc), lambda cb, rb: (0, cb)),
        ],
        out_specs=pl.BlockSpec((bre, bc), lambda cb, rb: (rb, cb)),
        out_shape=jax.ShapeDtypeStruct((n, d_dict), jnp.float32),
    )(x, W_enc, b_enc.reshape(1, d_dict))

    wdt = W_dec.T.astype(jnp.bfloat16)
    sparse, x_hat = pl.pallas_call(
        _select_decode_body,
        grid=(n // br2,),
        in_specs=[
            pl.BlockSpec((br2, d_dict), lambda i: (i, 0)),
            pl.BlockSpec((d_dict, d_model), lambda i: (0, 0)),
            pl.BlockSpec((1, d_model), lambda i: (0, 0)),
        ],
        out_specs=[
            pl.BlockSpec((br2, d_dict), lambda i: (i, 0)),
            pl.BlockSpec((br2, d_model), lambda i: (i, 0)),
        ],
        out_shape=[
            jax.ShapeDtypeStruct((n, d_dict), jnp.float32),
            jax.ShapeDtypeStruct((n, d_model), jnp.float32),
        ],
        input_output_aliases={0: 0},
    )(a, wdt, b_dec.reshape(1, d_model))
    return (x_hat, sparse)


# speedup vs baseline: 1.0620x; 1.0620x over previous
import jax
import jax.numpy as jnp
from jax.experimental import pallas as pl

_K = 32


def _encode_body(x_ref, w_ref, b_ref, a_ref):
    z = jax.lax.dot_general(
        x_ref[...], w_ref[...], (((1,), (1,)), ((), ())),
        preferred_element_type=jnp.float32)
    z = z + b_ref[...]
    a_ref[...] = jnp.where(z > 0.0, z, 0.0)


def _select_decode_body(a_ref, wdt_ref, bd_ref, sp_ref, xh_ref):
    a = a_ref[...]
    ai = jax.lax.bitcast_convert_type(a, jnp.int32)
    rows = a.shape[0]

    def bit_step(b, t):
        cand = t | jax.lax.shift_left(1, 30 - b)
        cnt = jnp.sum((ai >= cand).astype(jnp.int32), axis=1, keepdims=True)
        return jnp.where(cnt >= _K, cand, t)

    t = jax.lax.fori_loop(0, 31, bit_step, jnp.zeros((rows, 1), jnp.int32),
                          unroll=31)
    s = jnp.where(ai >= t, a, 0.0)
    sp_ref[...] = s
    xh = jax.lax.dot_general(
        s.astype(jnp.bfloat16), wdt_ref[...], (((1,), (0,)), ((), ())),
        preferred_element_type=jnp.float32)
    xh_ref[...] = xh + bd_ref[...]


def kernel(x, W_enc, b_enc, W_dec, b_dec):
    n, d_model = x.shape
    d_dict = W_enc.shape[0]
    bre = min(512, n)
    bc = min(2048, d_dict)
    br2 = min(256, n)

    a = pl.pallas_call(
        _encode_body,
        grid=(d_dict // bc, n // bre),
        in_specs=[
            pl.BlockSpec((bre, d_model), lambda cb, rb: (rb, 0)),
            pl.BlockSpec((bc, d_model), lambda cb, rb: (cb, 0)),
            pl.BlockSpec((1, bc), lambda cb, rb: (0, cb)),
        ],
        out_specs=pl.BlockSpec((bre, bc), lambda cb, rb: (rb, cb)),
        out_shape=jax.ShapeDtypeStruct((n, d_dict), jnp.float32),
    )(x, W_enc, b_enc.reshape(1, d_dict))

    wdt = W_dec.T.astype(jnp.bfloat16)
    sparse, x_hat = pl.pallas_call(
        _select_decode_body,
        grid=(n // br2,),
        in_specs=[
            pl.BlockSpec((br2, d_dict), lambda i: (i, 0)),
            pl.BlockSpec((d_dict, d_model), lambda i: (0, 0)),
            pl.BlockSpec((1, d_model), lambda i: (0, 0)),
        ],
        out_specs=[
            pl.BlockSpec((br2, d_dict), lambda i: (i, 0)),
            pl.BlockSpec((br2, d_model), lambda i: (i, 0)),
        ],
        out_shape=[
            jax.ShapeDtypeStruct((n, d_dict), jnp.float32),
            jax.ShapeDtypeStruct((n, d_model), jnp.float32),
        ],
        input_output_aliases={0: 0},
    )(a, wdt, b_dec.reshape(1, d_model))
    return (x_hat, sparse)
